# baseline (speedup 1.0000x reference)
import functools

import jax
import jax.numpy as jnp
from jax import lax
from jax.experimental import pallas as pl
from jax.experimental.pallas import tpu as pltpu
from jax.experimental.pallas import tpu_sc as plsc

NC = 2
NS = 16
L = 16
BLK = 128
FC = 64
RB = 1000


def _make_deg(n, ept):
    mesh = plsc.VectorSubcoreMesh(core_axis_name="c", subcore_axis_name="s")

    @functools.partial(
        pl.kernel,
        out_type=jax.ShapeDtypeStruct((NC * NS, n), jnp.float32),
        mesh=mesh,
        compiler_params=pltpu.CompilerParams(needs_layout_passes=False, use_tc_tiling_on_sc=False),
        scratch_types=[
            pltpu.VMEM((ept,), jnp.int32),
            pltpu.VMEM((ept,), jnp.float32),
            pltpu.VMEM((n,), jnp.float32),
        ],
    )
    def deg_kernel(dst_hbm, ew_hbm, out_hbm, dst_v, ew_v, deg_v):
        c = lax.axis_index("c")
        s = lax.axis_index("s")
        wid = c * NS + s
        pltpu.sync_copy(dst_hbm.at[wid], dst_v)
        pltpu.sync_copy(ew_hbm.at[wid], ew_v)

        zeros = jnp.zeros((L,), jnp.float32)

        def zero_body(i, _):
            deg_v[pl.ds(i * L, L)] = zeros
            return 0

        lax.fori_loop(0, n // L, zero_body, 0)

        def edge_body(g, _):
            idx = dst_v[pl.ds(g * L, L)]
            w = ew_v[pl.ds(g * L, L)]
            plsc.addupdate_scatter(deg_v, [idx], w)
            return 0

        lax.fori_loop(0, ept // L, edge_body, 0)
        pltpu.sync_copy(deg_v, out_hbm.at[wid])

    return deg_kernel


def _make_agg(n, n_chunks, nb):
    rows_per_tile = n // NS
    zr = rows_per_tile // 5
    mesh = plsc.VectorSubcoreMesh(core_axis_name="c", subcore_axis_name="s")

    @functools.partial(
        pl.kernel,
        out_type=jax.ShapeDtypeStruct((n_chunks * NC, n, FC), jnp.float32),
        mesh=mesh,
        compiler_params=pltpu.CompilerParams(needs_layout_passes=False, use_tc_tiling_on_sc=False),
        scratch_types=[
            pltpu.VMEM((nb, BLK), jnp.int32),
            pltpu.VMEM((nb, BLK), jnp.int32),
            pltpu.VMEM((nb, BLK), jnp.float32),
            pltpu.VMEM((4, BLK, FC), jnp.float32),
            pltpu.VMEM((zr, FC), jnp.float32),
            pltpu.VMEM_SHARED((n, FC), jnp.float32),
            [pltpu.SemaphoreType.DMA] * 4,
            [pltpu.SemaphoreType.DMA] * 4,
        ],
    )
    def agg_kernel(ht_hbm, src_hbm, dst_hbm, ew_hbm, out_hbm,
                   src_v, dst_v, ew_v, rows_v, zb_v, acc_sh, gsems, ssems):
        c = lax.axis_index("c")
        s = lax.axis_index("s")
        wid = c * NS + s

        pltpu.sync_copy(src_hbm.at[wid], src_v)
        pltpu.sync_copy(dst_hbm.at[wid], dst_v)
        pltpu.sync_copy(ew_hbm.at[wid], ew_v)

        zeros = jnp.zeros((L,), jnp.float32)

        @plsc.parallel_loop(0, zr)
        def _(r):
            for j in range(FC // L):
                zb_v[r, pl.ds(j * L, L)] = zeros

        for ci0 in range(n_chunks):
            ci = (ci0 + c * (n_chunks // 2)) % n_chunks
            ht_c = ht_hbm.at[ci]
            for j in range(rows_per_tile // zr):
                pltpu.sync_copy(zb_v, acc_sh.at[pl.ds(s * rows_per_tile + j * zr, zr)])
            plsc.subcore_barrier()

            for b in range(2):
                pltpu.async_copy(ht_c.at[src_v.at[b]], rows_v.at[b], gsems[b])

            def outer(i, _):
                for b in range(4):
                    k = i * 4 + b
                    bp = (b + 2) % 4

                    pltpu.make_async_copy(
                        ht_c.at[src_v.at[k]], rows_v.at[b], gsems[b]).wait()

                    @plsc.parallel_loop(0, BLK // L, unroll=2)
                    def _(g):
                        wv = ew_v[k, pl.ds(g * L, L)]
                        for t in range(L):
                            w = wv[t]
                            e = g * L + t
                            for j in range(FC // L):
                                rows_v[b, e, pl.ds(j * L, L)] = (
                                    rows_v[b, e, pl.ds(j * L, L)] * w)

                    pltpu.async_copy(rows_v.at[b], acc_sh.at[dst_v.at[k]],
                                     ssems[b], add=True)

                    @pl.when(k + 2 < nb)
                    def _():
                        @pl.when(k >= 2)
                        def _():
                            pltpu.make_async_copy(
                                rows_v.at[bp], acc_sh.at[dst_v.at[k]],
                                ssems[bp]).wait()

                        pltpu.async_copy(
                            ht_c.at[src_v.at[k + 2]], rows_v.at[bp], gsems[bp])
                return 0

            lax.fori_loop(0, nb // 4, outer, 0)
            for b in range(4):
                pltpu.make_async_copy(
                    rows_v.at[b], acc_sh.at[dst_v.at[0]], ssems[b]).wait()
            plsc.subcore_barrier()
            pltpu.sync_copy(
                acc_sh.at[pl.ds(s * rows_per_tile, rows_per_tile)],
                out_hbm.at[ci * NC + c, pl.ds(s * rows_per_tile, rows_per_tile)])

    return agg_kernel


def _mm1(degp, x, W1, n, din, dh):
    nchunk = dh // FC

    def body(degp_ref, x_ref, w1_ref, ht_ref, dis_ref):
        degsum = jnp.sum(degp_ref[0], axis=0) + 1.0
        dis = lax.rsqrt(degsum)[:, None]
        dis_ref[...] = dis
        ht_ref[0] = dis * jnp.dot(x_ref[...], w1_ref[0],
                                  preferred_element_type=jnp.float32)

    return pl.pallas_call(
        body,
        grid=(n // RB, nchunk),
        in_specs=[
            pl.BlockSpec((1, NC * NS, RB), lambda i, c: (i, 0, 0)),
            pl.BlockSpec((RB, din), lambda i, c: (i, 0)),
            pl.BlockSpec((1, din, FC), lambda i, c: (c, 0, 0)),
        ],
        out_specs=[
            pl.BlockSpec((1, RB, FC), lambda i, c: (c, i, 0)),
            pl.BlockSpec((RB, 1), lambda i, c: (i, 0)),
        ],
        out_shape=[
            jax.ShapeDtypeStruct((nchunk, n, FC), jnp.float32),
            jax.ShapeDtypeStruct((n, 1), jnp.float32),
        ],
    )(degp, x, W1)


def _mm2(agg1, ht, dis, b1r, W2r, n, dh, dout):
    kc = dh // FC
    oc = dout // FC

    def body(agg_ref, ht_ref, dis_ref, b1_ref, w2_ref, out_ref):
        c = pl.program_id(1)
        dis = dis_ref[...]
        a = agg_ref[0, 0] + agg_ref[0, 1] + ht_ref[0]
        a = jnp.maximum(dis * a + b1_ref[0], 0.0)

        @pl.when(c == 0)
        def _():
            out_ref[...] = jnp.zeros_like(out_ref)

        for j in range(oc):
            out_ref[j] += jnp.dot(a, w2_ref[0, :, j, :],
                                  preferred_element_type=jnp.float32)

        @pl.when(c == kc - 1)
        def _():
            out_ref[...] = dis[None] * out_ref[...]

    return pl.pallas_call(
        body,
        grid=(n // RB, kc),
        in_specs=[
            pl.BlockSpec((1, NC, RB, FC), lambda i, c: (c, 0, i, 0)),
            pl.BlockSpec((1, RB, FC), lambda i, c: (c, i, 0)),
            pl.BlockSpec((RB, 1), lambda i, c: (i, 0)),
            pl.BlockSpec((1, 1, FC), lambda i, c: (c, 0, 0)),
            pl.BlockSpec((1, FC, oc, FC), lambda i, c: (c, 0, 0, 0)),
        ],
        out_specs=pl.BlockSpec((oc, RB, FC), lambda i, c: (0, i, 0)),
        out_shape=jax.ShapeDtypeStruct((oc, n, FC), jnp.float32),
    )(agg1, ht, dis, b1r, W2r)


def _fin(agg2, ht2, dis, b2r, n, dout):
    oc = dout // FC

    def body(agg_ref, ht_ref, dis_ref, b2_ref, out_ref):
        dis = dis_ref[...]
        for c in range(oc):
            out_ref[:, c] = (dis * (agg_ref[c, 0] + agg_ref[c, 1] + ht_ref[c])
                             + b2_ref[c])

    return pl.pallas_call(
        body,
        grid=(n // RB,),
        in_specs=[
            pl.BlockSpec((oc, NC, RB, FC), lambda i: (0, 0, i, 0)),
            pl.BlockSpec((oc, RB, FC), lambda i: (0, i, 0)),
            pl.BlockSpec((RB, 1), lambda i: (i, 0)),
            pl.BlockSpec((oc, 1, FC), lambda i: (0, 0, 0)),
        ],
        out_specs=pl.BlockSpec((RB, oc, FC), lambda i: (i, 0, 0)),
        out_shape=jax.ShapeDtypeStruct((n, oc, FC), jnp.float32),
    )(agg2, ht2, dis, b2r)


def kernel(x, edge_index, edge_weight, W1, b1, W2, b2):
    n, din = x.shape
    dh = W1.shape[1]
    dout = W2.shape[1]
    e = edge_weight.shape[0]

    ntiles = NC * NS
    quantum = ntiles * BLK
    e_pad = ((e + quantum - 1) // quantum) * quantum
    pad = e_pad - e
    src = jnp.concatenate([edge_index[0], jnp.zeros((pad,), jnp.int32)])
    dst = jnp.concatenate([edge_index[1], jnp.zeros((pad,), jnp.int32)])
    ew = jnp.concatenate([edge_weight, jnp.zeros((pad,), jnp.float32)])

    ept = e_pad // ntiles
    nb = ept // BLK
    src_r = src.reshape(ntiles, nb, BLK)
    dst_r = dst.reshape(ntiles, nb, BLK)
    ew_r = ew.reshape(ntiles, nb, BLK)

    degp = _make_deg(n, ept)(dst.reshape(ntiles, ept), ew.reshape(ntiles, ept))
    degp = degp.reshape(ntiles, n // RB, RB).swapaxes(0, 1)
    W1r = W1.reshape(din, dh // FC, FC).transpose(1, 0, 2)
    ht, dis = _mm1(degp, x, W1r, n, din, dh)

    agg1 = _make_agg(n, dh // FC, nb)(ht, src_r, dst_r, ew_r)
    agg1 = agg1.reshape(dh // FC, NC, n, FC)

    ht2 = _mm2(agg1, ht, dis, b1.reshape(dh // FC, 1, FC),
               W2.reshape(dh // FC, FC, dout // FC, FC), n, dh, dout)

    agg2 = _make_agg(n, dout // FC, nb)(ht2, src_r, dst_r, ew_r)
    agg2 = agg2.reshape(dout // FC, NC, n, FC)

    out = _fin(agg2, ht2, dis, b2.reshape(dout // FC, 1, FC), n, dout)
    return out.reshape(n, dout)


# speedup vs baseline: 1.3332x; 1.3332x over previous
import functools

import jax
import jax.numpy as jnp
from jax import lax
from jax.experimental import pallas as pl
from jax.experimental.pallas import tpu as pltpu
from jax.experimental.pallas import tpu_sc as plsc

NC = 2
NS = 16
L = 16
BLK = 128
FC = 64
RB = 1000


def _make_deg(n, ept):
    mesh = plsc.VectorSubcoreMesh(core_axis_name="c", subcore_axis_name="s")

    @functools.partial(
        pl.kernel,
        out_type=jax.ShapeDtypeStruct((NC * NS, n), jnp.float32),
        mesh=mesh,
        compiler_params=pltpu.CompilerParams(needs_layout_passes=False, use_tc_tiling_on_sc=False),
        scratch_types=[
            pltpu.VMEM((ept,), jnp.int32),
            pltpu.VMEM((ept,), jnp.float32),
            pltpu.VMEM((n,), jnp.float32),
        ],
    )
    def deg_kernel(dst_hbm, ew_hbm, out_hbm, dst_v, ew_v, deg_v):
        c = lax.axis_index("c")
        s = lax.axis_index("s")
        wid = c * NS + s
        pltpu.sync_copy(dst_hbm.at[wid], dst_v)
        pltpu.sync_copy(ew_hbm.at[wid], ew_v)

        zeros = jnp.zeros((L,), jnp.float32)

        def zero_body(i, _):
            deg_v[pl.ds(i * L, L)] = zeros
            return 0

        lax.fori_loop(0, n // L, zero_body, 0)

        def edge_body(g, _):
            idx = dst_v[pl.ds(g * L, L)]
            w = ew_v[pl.ds(g * L, L)]
            plsc.addupdate_scatter(deg_v, [idx], w)
            return 0

        lax.fori_loop(0, ept // L, edge_body, 0)
        pltpu.sync_copy(deg_v, out_hbm.at[wid])

    return deg_kernel


def _make_agg(n, n_chunks, nb):
    rows_per_tile = n // NS
    zr = rows_per_tile // 5
    mesh = plsc.VectorSubcoreMesh(core_axis_name="c", subcore_axis_name="s")

    @functools.partial(
        pl.kernel,
        out_type=jax.ShapeDtypeStruct((n_chunks * NC, n, FC), jnp.float32),
        mesh=mesh,
        compiler_params=pltpu.CompilerParams(needs_layout_passes=False, use_tc_tiling_on_sc=False),
        scratch_types=[
            pltpu.VMEM((nb, BLK), jnp.int32),
            pltpu.VMEM((nb, BLK), jnp.int32),
            pltpu.VMEM((nb, BLK), jnp.float32),
            pltpu.VMEM((4, BLK, FC), jnp.bfloat16),
            pltpu.VMEM((4, BLK, FC), jnp.float32),
            pltpu.VMEM((zr, FC), jnp.float32),
            pltpu.VMEM_SHARED((n, FC), jnp.float32),
            [pltpu.SemaphoreType.DMA] * 4,
            [pltpu.SemaphoreType.DMA] * 4,
        ],
    )
    def agg_kernel(ht_hbm, src_hbm, dst_hbm, ew_hbm, out_hbm,
                   src_v, dst_v, ew_v, rows_g, rows_s, zb_v, acc_sh,
                   gsems, ssems):
        c = lax.axis_index("c")
        s = lax.axis_index("s")
        wid = c * NS + s

        pltpu.sync_copy(src_hbm.at[wid], src_v)
        pltpu.sync_copy(dst_hbm.at[wid], dst_v)
        pltpu.sync_copy(ew_hbm.at[wid], ew_v)

        zeros = jnp.zeros((L,), jnp.float32)

        @plsc.parallel_loop(0, zr)
        def _(r):
            for j in range(FC // L):
                zb_v[r, pl.ds(j * L, L)] = zeros

        def chunk_body(ci0, _):
            ci = (ci0 + c * (n_chunks // 2)) % n_chunks
            ht_c = ht_hbm.at[ci]
            for j in range(rows_per_tile // zr):
                pltpu.sync_copy(zb_v, acc_sh.at[pl.ds(s * rows_per_tile + j * zr, zr)])
            plsc.subcore_barrier()

            for b in range(4):
                pltpu.async_copy(ht_c.at[src_v.at[b]], rows_g.at[b], gsems[b])

            def outer(i, _):
                for b in range(4):
                    k = i * 4 + b

                    pltpu.make_async_copy(
                        ht_c.at[src_v.at[k]], rows_g.at[b], gsems[b]).wait()

                    @pl.when(k >= 4)
                    def _():
                        pltpu.make_async_copy(
                            rows_s.at[b], acc_sh.at[dst_v.at[k]],
                            ssems[b]).wait()

                    @plsc.parallel_loop(0, BLK // L, unroll=2)
                    def _(g):
                        wv = ew_v[k, pl.ds(g * L, L)]
                        for t in range(L):
                            w = wv[t]
                            e = g * L + t
                            for p in range(FC // 32):
                                v = rows_g[b, e, pl.ds(p * 32, 32)]
                                lo, hi = plsc.unpack(
                                    v, format=plsc.PackFormat.INTERLEAVED,
                                    preferred_element_type=jnp.float32)
                                rows_s[b, e, pl.ds(p * 32, L)] = lo * w
                                rows_s[b, e, pl.ds(p * 32 + L, L)] = hi * w

                    pltpu.async_copy(rows_s.at[b], acc_sh.at[dst_v.at[k]],
                                     ssems[b], add=True)

                    @pl.when(k + 4 < nb)
                    def _():
                        pltpu.async_copy(
                            ht_c.at[src_v.at[k + 4]], rows_g.at[b], gsems[b])
                return 0

            lax.fori_loop(0, nb // 4, outer, 0)
            for b in range(4):
                pltpu.make_async_copy(
                    rows_s.at[b], acc_sh.at[dst_v.at[0]], ssems[b]).wait()
            plsc.subcore_barrier()
            pltpu.sync_copy(
                acc_sh.at[pl.ds(s * rows_per_tile, rows_per_tile)],
                out_hbm.at[ci * NC + c, pl.ds(s * rows_per_tile, rows_per_tile)])
            return 0

        lax.fori_loop(0, n_chunks, chunk_body, 0)

    return agg_kernel


def _bf_interleave(ht):
    nch, n, fc = ht.shape
    r = ht.reshape(nch, n, fc // 32, 2, L)
    r = r.transpose(0, 1, 2, 4, 3)
    return r.reshape(nch, n, fc).astype(jnp.bfloat16)


def _mm1(degp, x, W1, n, din, dh):
    nchunk = dh // FC

    def body(degp_ref, x_ref, w1_ref, ht_ref, dis_ref):
        degsum = jnp.sum(degp_ref[0], axis=0) + 1.0
        dis = lax.rsqrt(degsum)[:, None]
        dis_ref[...] = dis
        ht_ref[0] = dis * jnp.dot(x_ref[...], w1_ref[0],
                                  preferred_element_type=jnp.float32)

    return pl.pallas_call(
        body,
        grid=(n // RB, nchunk),
        in_specs=[
            pl.BlockSpec((1, NC * NS, RB), lambda i, c: (i, 0, 0)),
            pl.BlockSpec((RB, din), lambda i, c: (i, 0)),
            pl.BlockSpec((1, din, FC), lambda i, c: (c, 0, 0)),
        ],
        out_specs=[
            pl.BlockSpec((1, RB, FC), lambda i, c: (c, i, 0)),
            pl.BlockSpec((RB, 1), lambda i, c: (i, 0)),
        ],
        out_shape=[
            jax.ShapeDtypeStruct((nchunk, n, FC), jnp.float32),
            jax.ShapeDtypeStruct((n, 1), jnp.float32),
        ],
    )(degp, x, W1)


def _mm2(agg1, ht, dis, b1r, W2r, n, dh, dout):
    kc = dh // FC
    oc = dout // FC

    def body(agg_ref, ht_ref, dis_ref, b1_ref, w2_ref, out_ref):
        c = pl.program_id(1)
        dis = dis_ref[...]
        a = agg_ref[0, 0] + agg_ref[0, 1] + ht_ref[0]
        a = jnp.maximum(dis * a + b1_ref[0], 0.0)

        @pl.when(c == 0)
        def _():
            out_ref[...] = jnp.zeros_like(out_ref)

        for j in range(oc):
            out_ref[j] += jnp.dot(a, w2_ref[0, :, j, :],
                                  preferred_element_type=jnp.float32)

        @pl.when(c == kc - 1)
        def _():
            out_ref[...] = dis[None] * out_ref[...]

    return pl.pallas_call(
        body,
        grid=(n // RB, kc),
        in_specs=[
            pl.BlockSpec((1, NC, RB, FC), lambda i, c: (c, 0, i, 0)),
            pl.BlockSpec((1, RB, FC), lambda i, c: (c, i, 0)),
            pl.BlockSpec((RB, 1), lambda i, c: (i, 0)),
            pl.BlockSpec((1, 1, FC), lambda i, c: (c, 0, 0)),
            pl.BlockSpec((1, FC, oc, FC), lambda i, c: (c, 0, 0, 0)),
        ],
        out_specs=pl.BlockSpec((oc, RB, FC), lambda i, c: (0, i, 0)),
        out_shape=jax.ShapeDtypeStruct((oc, n, FC), jnp.float32),
    )(agg1, ht, dis, b1r, W2r)


def _fin(agg2, ht2, dis, b2r, n, dout):
    oc = dout // FC

    def body(agg_ref, ht_ref, dis_ref, b2_ref, out_ref):
        dis = dis_ref[...]
        for c in range(oc):
            out_ref[:, c] = (dis * (agg_ref[c, 0] + agg_ref[c, 1] + ht_ref[c])
                             + b2_ref[c])

    return pl.pallas_call(
        body,
        grid=(n // RB,),
        in_specs=[
            pl.BlockSpec((oc, NC, RB, FC), lambda i: (0, 0, i, 0)),
            pl.BlockSpec((oc, RB, FC), lambda i: (0, i, 0)),
            pl.BlockSpec((RB, 1), lambda i: (i, 0)),
            pl.BlockSpec((oc, 1, FC), lambda i: (0, 0, 0)),
        ],
        out_specs=pl.BlockSpec((RB, oc, FC), lambda i: (i, 0, 0)),
        out_shape=jax.ShapeDtypeStruct((n, oc, FC), jnp.float32),
    )(agg2, ht2, dis, b2r)


def kernel(x, edge_index, edge_weight, W1, b1, W2, b2):
    n, din = x.shape
    dh = W1.shape[1]
    dout = W2.shape[1]
    e = edge_weight.shape[0]

    ntiles = NC * NS
    quantum = ntiles * BLK
    e_pad = ((e + quantum - 1) // quantum) * quantum
    pad = e_pad - e
    src = jnp.concatenate([edge_index[0], jnp.zeros((pad,), jnp.int32)])
    dst = jnp.concatenate([edge_index[1], jnp.zeros((pad,), jnp.int32)])
    ew = jnp.concatenate([edge_weight, jnp.zeros((pad,), jnp.float32)])

    ept = e_pad // ntiles
    nb = ept // BLK
    src_r = src.reshape(ntiles, nb, BLK)
    dst_r = dst.reshape(ntiles, nb, BLK)
    ew_r = ew.reshape(ntiles, nb, BLK)

    degp = _make_deg(n, ept)(dst.reshape(ntiles, ept), ew.reshape(ntiles, ept))
    degp = degp.reshape(ntiles, n // RB, RB).swapaxes(0, 1)
    W1r = W1.reshape(din, dh // FC, FC).transpose(1, 0, 2)
    ht, dis = _mm1(degp, x, W1r, n, din, dh)

    agg1 = _make_agg(n, dh // FC, nb)(_bf_interleave(ht), src_r, dst_r, ew_r)
    agg1 = agg1.reshape(dh // FC, NC, n, FC)

    ht2 = _mm2(agg1, ht, dis, b1.reshape(dh // FC, 1, FC),
               W2.reshape(dh // FC, FC, dout // FC, FC), n, dh, dout)

    agg2 = _make_agg(n, dout // FC, nb)(_bf_interleave(ht2), src_r, dst_r, ew_r)
    agg2 = agg2.reshape(dout // FC, NC, n, FC)

    out = _fin(agg2, ht2, dis, b2.reshape(dout // FC, 1, FC), n, dout)
    return out.reshape(n, dout)
